# baseline (speedup 1.0000x reference)
import functools
import math

import jax
import jax.numpy as jnp
from jax import lax
from jax.experimental import pallas as pl
from jax.experimental.pallas import tpu as pltpu
from jax.experimental.pallas import tpu_sc as plsc

NC = 2
NS = 16
NW = NC * NS
LANES = 16
CH = 64
EPS = 1e-5


def _rsqrt(x):
    bits = lax.bitcast_convert_type(x, jnp.int32)
    y = lax.bitcast_convert_type(jnp.int32(0x5F3759DF) - (bits >> 1),
                                 jnp.float32)
    for _ in range(3):
        y = y * (1.5 - 0.5 * x * y * y)
    return y


_GATHER_DNUMS = lax.GatherDimensionNumbers(
    offset_dims=(), collapsed_slice_dims=(0,), start_index_map=(0,))


def _shuffle(x, idx):
    return lax.gather(x, idx[:, None], _GATHER_DNUMS, (1,),
                      mode=lax.GatherScatterMode.PROMISE_IN_BOUNDS)


def _lane_sum(x):
    idx = lax.iota(jnp.int32, LANES)
    for k in (1, 2, 4, 8):
        x = x + _shuffle(x, idx ^ k)
    return x


def _positional_table(seq_len, d_b):
    pos = jnp.arange(seq_len, dtype=jnp.float32)[:, None]
    div = jnp.exp(
        jnp.arange(0, d_b, 2, dtype=jnp.float32) * (-(math.log(10000.0) / d_b)))
    pe = jnp.zeros((seq_len, d_b), dtype=jnp.float32)
    pe = pe.at[:, 0::2].set(jnp.sin(pos * div))
    pe = pe.at[:, 1::2].set(jnp.cos(pos * div[: d_b // 2 + d_b % 2]))
    return pe


@functools.lru_cache(maxsize=None)
def _build(n_tok, seq_len, d):
    dm = 3 * d
    nch = n_tok // (NW * CH)
    ncv = d // LANES

    mesh = plsc.VectorSubcoreMesh(core_axis_name="c", subcore_axis_name="s")

    @functools.partial(
        pl.kernel,
        out_type=jax.ShapeDtypeStruct((n_tok, dm), jnp.float32),
        mesh=mesh,
        scratch_types=[
            pltpu.VMEM((CH,), jnp.int32),
            pltpu.VMEM((CH, d), jnp.float32),
            pltpu.VMEM((CH, d), jnp.float32),
            pltpu.VMEM((CH, d), jnp.float32),
            pltpu.VMEM((CH, d), jnp.float32),
            pltpu.VMEM((dm,), jnp.float32),
            pltpu.VMEM((dm,), jnp.float32),
            pltpu.VMEM((CH, LANES), jnp.float32),
            pltpu.VMEM((CH, LANES), jnp.float32),
            pltpu.SemaphoreType.DMA,
        ],
    )
    def launch(tok_hbm, wa_hbm, wf_hbm, wb_hbm, gamma_hbm, beta_hbm, pe_hbm,
               out_hbm, idx_v, a_v, f_v, b_v, pe_v, g_v, bb_v,
               mu_v, rs_v, sem):
        wid = lax.axis_index("s") * NC + lax.axis_index("c")
        base = wid * (n_tok // NW)

        pltpu.sync_copy(gamma_hbm, g_v)
        pltpu.sync_copy(beta_hbm, bb_v)

        sections = ((a_v, False), (f_v, False), (b_v, True))

        def chunk_body(c, _):
            cb = base + c * CH
            pltpu.sync_copy(tok_hbm.at[pl.ds(cb, CH)], idx_v)
            cp_a = pltpu.async_copy(wa_hbm.at[idx_v], a_v, sem)
            cp_f = pltpu.async_copy(wf_hbm.at[idx_v], f_v, sem)
            cp_b = pltpu.async_copy(wb_hbm.at[idx_v], b_v, sem)
            pos = lax.rem(cb, seq_len)
            pltpu.sync_copy(pe_hbm.at[pl.ds(pos, CH)], pe_v)
            cp_a.wait()
            cp_f.wait()
            cp_b.wait()

            def p1(t, _):
                s0 = jnp.zeros((LANES,), jnp.float32)
                q0 = jnp.zeros((LANES,), jnp.float32)
                for buf, use_pe in sections:
                    for j in range(ncv):
                        x = buf[t, pl.ds(j * LANES, LANES)]
                        if use_pe:
                            x = x + pe_v[t, pl.ds(j * LANES, LANES)]
                        s0 = s0 + x
                        q0 = q0 + x * x
                mu = _lane_sum(s0) * (1.0 / dm)
                var = _lane_sum(q0) * (1.0 / dm) - mu * mu
                mu_v[t, pl.ds(0, LANES)] = mu
                rs_v[t, pl.ds(0, LANES)] = _rsqrt(var + EPS)
                return 0

            lax.fori_loop(0, CH, p1, 0)

            def p2(t, _):
                m = mu_v[t, pl.ds(0, LANES)]
                r = rs_v[t, pl.ds(0, LANES)]
                for sec, (buf, use_pe) in enumerate(sections):
                    for j in range(ncv):
                        col = sec * d + j * LANES
                        x = buf[t, pl.ds(j * LANES, LANES)]
                        if use_pe:
                            x = x + pe_v[t, pl.ds(j * LANES, LANES)]
                        gs = g_v[pl.ds(col, LANES)] * r
                        buf[t, pl.ds(j * LANES, LANES)] = (
                            (x - m) * gs + bb_v[pl.ds(col, LANES)])
                return 0

            lax.fori_loop(0, CH, p2, 0)

            for sec, (buf, _unused) in enumerate(sections):
                pltpu.sync_copy(
                    buf, out_hbm.at[pl.ds(cb, CH), pl.ds(sec * d, d)])
            return 0

        lax.fori_loop(0, nch, chunk_body, 0)

    return launch


def kernel(tokens, Wa, Wf, Wb, gamma, beta):
    b, s = tokens.shape
    d = Wa.shape[1]
    tok = tokens.reshape(-1).astype(jnp.int32)
    pe = _positional_table(s, Wb.shape[1])
    out = _build(b * s, s, d)(tok, Wa, Wf, Wb, gamma, beta, pe)
    return out.reshape(b, s, 3 * d)


# speedup vs baseline: 1.0480x; 1.0480x over previous
import functools
import math

import jax
import jax.numpy as jnp
from jax import lax
from jax.experimental import pallas as pl
from jax.experimental.pallas import tpu as pltpu
from jax.experimental.pallas import tpu_sc as plsc

NC = 2
NS = 16
NW = NC * NS
LANES = 16
CH = 32
EPS = 1e-5


def _rsqrt(x):
    bits = lax.bitcast_convert_type(x, jnp.int32)
    y = lax.bitcast_convert_type(jnp.int32(0x5F3759DF) - (bits >> 1),
                                 jnp.float32)
    for _ in range(3):
        y = y * (1.5 - 0.5 * x * y * y)
    return y


_GATHER_DNUMS = lax.GatherDimensionNumbers(
    offset_dims=(), collapsed_slice_dims=(0,), start_index_map=(0,))


def _shuffle(x, idx):
    return lax.gather(x, idx[:, None], _GATHER_DNUMS, (1,),
                      mode=lax.GatherScatterMode.PROMISE_IN_BOUNDS)


def _lane_sum(x):
    idx = lax.iota(jnp.int32, LANES)
    for k in (1, 2, 4, 8):
        x = x + _shuffle(x, idx ^ k)
    return x


def _positional_table(seq_len, d_b):
    pos = jnp.arange(seq_len, dtype=jnp.float32)[:, None]
    div = jnp.exp(
        jnp.arange(0, d_b, 2, dtype=jnp.float32) * (-(math.log(10000.0) / d_b)))
    pe = jnp.zeros((seq_len, d_b), dtype=jnp.float32)
    pe = pe.at[:, 0::2].set(jnp.sin(pos * div))
    pe = pe.at[:, 1::2].set(jnp.cos(pos * div[: d_b // 2 + d_b % 2]))
    return pe


@functools.lru_cache(maxsize=None)
def _build(n_tok, seq_len, d):
    dm = 3 * d
    tok_w = n_tok // NW
    nch = tok_w // CH
    ncv = d // LANES

    mesh = plsc.VectorSubcoreMesh(core_axis_name="c", subcore_axis_name="s")

    buf_t = pltpu.VMEM((CH, d), jnp.float32)

    @functools.partial(
        pl.kernel,
        out_type=jax.ShapeDtypeStruct((n_tok, dm), jnp.float32),
        mesh=mesh,
        scratch_types=[
            pltpu.VMEM((tok_w,), jnp.int32),
            [buf_t, buf_t],
            [buf_t, buf_t],
            [buf_t, buf_t],
            [buf_t, buf_t],
            pltpu.VMEM((dm,), jnp.float32),
            pltpu.VMEM((dm,), jnp.float32),
            pltpu.VMEM((CH, LANES), jnp.float32),
            pltpu.VMEM((CH, LANES), jnp.float32),
            [pltpu.SemaphoreType.DMA] * 2,
            [pltpu.SemaphoreType.DMA] * 2,
        ],
    )
    def launch(tok_hbm, wa_hbm, wf_hbm, wb_hbm, gamma_hbm, beta_hbm, pe_hbm,
               out_hbm, tid_v, a_v, f_v, b_v, pe_v, g_v, bb_v,
               mu_v, rs_v, gsem, osem):
        wid = lax.axis_index("s") * NC + lax.axis_index("c")
        base = wid * tok_w

        pltpu.sync_copy(tok_hbm.at[pl.ds(base, tok_w)], tid_v)
        pltpu.sync_copy(gamma_hbm, g_v)
        pltpu.sync_copy(beta_hbm, bb_v)

        def issue(c, s):
            idx = tid_v.at[pl.ds(c * CH, CH)]
            pos = lax.rem(base + c * CH, seq_len)
            return (
                pltpu.async_copy(wa_hbm.at[idx], a_v[s], gsem[s]),
                pltpu.async_copy(wf_hbm.at[idx], f_v[s], gsem[s]),
                pltpu.async_copy(wb_hbm.at[idx], b_v[s], gsem[s]),
                pltpu.async_copy(pe_hbm.at[pl.ds(pos, CH)], pe_v[s], gsem[s]),
            )

        def writeback(c, s):
            cb = base + c * CH
            return tuple(
                pltpu.async_copy(
                    buf, out_hbm.at[pl.ds(cb, CH), pl.ds(sec * d, d)], osem[s])
                for sec, buf in enumerate((a_v[s], f_v[s], b_v[s])))

        def compute(s):
            sections = ((a_v[s], False), (f_v[s], False), (b_v[s], True))

            def p1(t, _):
                s0 = jnp.zeros((LANES,), jnp.float32)
                q0 = jnp.zeros((LANES,), jnp.float32)
                for buf, use_pe in sections:
                    for j in range(ncv):
                        x = buf[t, pl.ds(j * LANES, LANES)]
                        if use_pe:
                            x = x + pe_v[s][t, pl.ds(j * LANES, LANES)]
                        s0 = s0 + x
                        q0 = q0 + x * x
                mu = _lane_sum(s0) * (1.0 / dm)
                var = _lane_sum(q0) * (1.0 / dm) - mu * mu
                mu_v[t, pl.ds(0, LANES)] = mu
                rs_v[t, pl.ds(0, LANES)] = _rsqrt(var + EPS)
                return 0

            lax.fori_loop(0, CH, p1, 0)

            def p2(t, _):
                m = mu_v[t, pl.ds(0, LANES)]
                r = rs_v[t, pl.ds(0, LANES)]
                for sec, (buf, use_pe) in enumerate(sections):
                    for j in range(ncv):
                        col = sec * d + j * LANES
                        x = buf[t, pl.ds(j * LANES, LANES)]
                        if use_pe:
                            x = x + pe_v[s][t, pl.ds(j * LANES, LANES)]
                        gs = g_v[pl.ds(col, LANES)] * r
                        buf[t, pl.ds(j * LANES, LANES)] = (
                            (x - m) * gs + bb_v[pl.ds(col, LANES)])
                return 0

            lax.fori_loop(0, CH, p2, 0)

        pend_in = [None, None]
        pend_out = [None, None]
        pend_in[0] = issue(0, 0)
        for c in range(nch):
            s = c & 1
            ns = 1 - s
            if c + 1 < nch:
                if pend_out[ns] is not None:
                    for dsc in pend_out[ns]:
                        dsc.wait()
                    pend_out[ns] = None
                pend_in[ns] = issue(c + 1, ns)
            for dsc in pend_in[s]:
                dsc.wait()
            compute(s)
            pend_out[s] = writeback(c, s)
        for s in (0, 1):
            if pend_out[s] is not None:
                for dsc in pend_out[s]:
                    dsc.wait()

    return launch


def kernel(tokens, Wa, Wf, Wb, gamma, beta):
    b, s = tokens.shape
    d = Wa.shape[1]
    tok = tokens.reshape(-1).astype(jnp.int32)
    pe = _positional_table(s, Wb.shape[1])
    out = _build(b * s, s, d)(tok, Wa, Wf, Wb, gamma, beta, pe)
    return out.reshape(b, s, 3 * d)
